# baseline (speedup 1.0000x reference)
import functools

import jax
import jax.numpy as jnp
from jax import lax
from jax.experimental import pallas as pl
from jax.experimental.pallas import tpu as pltpu
from jax.experimental.pallas import tpu_sc as plsc

N = 10000
NP = 10240
E = 160000
EP = 163840
PAD_ROW = 10000
D_IN = 256
D_HID = 512
D_OUT = 32
K = 128
NB_A = EP // 16 // K
NB_E = EP // 32 // K
RPS = NP // 16

_mesh = plsc.VectorSubcoreMesh(core_axis_name="c", subcore_axis_name="s",
                               num_cores=2, num_subcores=16)

f32 = jnp.float32


def _sc_hist(dst32, zeros128, ones128):
  @functools.partial(
      pl.kernel,
      out_type=jax.ShapeDtypeStruct((2, NP, 128), f32),
      mesh=_mesh,
      scratch_types=[
          pltpu.VMEM((NB_E, K), jnp.int32),
          pltpu.VMEM((K, 128), f32),
          pltpu.VMEM_SHARED((NP, 128), f32),
      ],
  )
  def hist(dst_hbm, z_hbm, ones_hbm, out, dst_v, ones_v, acc):
    c = lax.axis_index("c")
    s = lax.axis_index("s")
    w = s * 2 + c
    rows = pl.ds(s * RPS, RPS)
    pltpu.sync_copy(dst_hbm.at[w], dst_v)
    pltpu.sync_copy(ones_hbm, ones_v)
    pltpu.sync_copy(z_hbm.at[rows], acc.at[rows])
    plsc.subcore_barrier()

    def body(b, carry):
      pltpu.sync_copy(ones_v, acc.at[dst_v.at[b]], add=True)
      return carry

    lax.fori_loop(0, NB_E, body, 0)
    plsc.subcore_barrier()
    pltpu.sync_copy(acc.at[rows], out.at[c, rows])

  h = hist(dst32, zeros128, ones128)
  return h[0], h[1]


HB = 40


def _agg_chunk(t_hbm, init_hbm, o_hbm, acc, src_hbm, dst_hbm, row_i,
               src_v, dst_v, buf0, buf1, sem0, sem1, ssem0, ssem1, s, nb):
  rows = pl.ds(s * RPS, RPS)
  pltpu.sync_copy(init_hbm.at[rows], acc.at[rows])
  plsc.subcore_barrier()

  for h in range(nb // HB):
    pltpu.sync_copy(src_hbm.at[row_i, pl.ds(h * HB, HB)], src_v)
    pltpu.sync_copy(dst_hbm.at[row_i, pl.ds(h * HB, HB)], dst_v)
    pltpu.async_copy(t_hbm.at[src_v.at[0]], buf0, sem0)
    pltpu.async_copy(t_hbm.at[src_v.at[1]], buf1, sem1)

    def body(i, carry):
      b = 2 * i
      pltpu.make_async_copy(t_hbm.at[src_v.at[b]], buf0, sem0).wait()
      pltpu.async_copy(buf0, acc.at[dst_v.at[b]], ssem0, add=True)
      pltpu.make_async_copy(t_hbm.at[src_v.at[b + 1]], buf1, sem1).wait()
      pltpu.async_copy(buf1, acc.at[dst_v.at[b + 1]], ssem1, add=True)
      pltpu.make_async_copy(buf0, acc.at[dst_v.at[b]], ssem0).wait()

      @pl.when(b + 2 < HB)
      def _():
        pltpu.async_copy(t_hbm.at[src_v.at[b + 2]], buf0, sem0)

      pltpu.make_async_copy(buf1, acc.at[dst_v.at[b + 1]], ssem1).wait()

      @pl.when(b + 3 < HB)
      def _():
        pltpu.async_copy(t_hbm.at[src_v.at[b + 3]], buf1, sem1)

      return carry

    lax.fori_loop(0, HB // 2, body, 0)

  plsc.subcore_barrier()
  pltpu.sync_copy(acc.at[rows], o_hbm.at[rows])


def _sc_agg_cols(tables, src16, dst16):
  n = len(tables)
  per_core = n // 2

  @functools.partial(
      pl.kernel,
      out_type=[jax.ShapeDtypeStruct((NP, 128), f32)] * n,
      mesh=_mesh,
      scratch_types=[
          pltpu.VMEM((HB, K), jnp.int32),
          pltpu.VMEM((HB, K), jnp.int32),
          pltpu.VMEM((K, 128), f32),
          pltpu.VMEM((K, 128), f32),
          pltpu.VMEM_SHARED((NP, 128), f32),
          pltpu.SemaphoreType.DMA,
          pltpu.SemaphoreType.DMA,
          pltpu.SemaphoreType.DMA,
          pltpu.SemaphoreType.DMA,
      ],
  )
  def agg(*refs):
    t_refs = refs[:n]
    src_hbm, dst_hbm = refs[n], refs[n + 1]
    o_refs = refs[n + 2:2 * n + 2]
    src_v, dst_v, buf0, buf1, acc, sem0, sem1, ssem0, ssem1 = refs[2 * n + 2:]
    c = lax.axis_index("c")
    s = lax.axis_index("s")

    @pl.when(c == 0)
    def _():
      for j in range(per_core):
        _agg_chunk(t_refs[j], t_refs[j], o_refs[j], acc, src_hbm, dst_hbm, s,
                   src_v, dst_v, buf0, buf1, sem0, sem1, ssem0, ssem1, s,
                   NB_A)
        if j + 1 < per_core:
          plsc.subcore_barrier()

    @pl.when(c == 1)
    def _():
      for j in range(per_core, n):
        _agg_chunk(t_refs[j], t_refs[j], o_refs[j], acc, src_hbm, dst_hbm, s,
                   src_v, dst_v, buf0, buf1, sem0, sem1, ssem0, ssem1, s,
                   NB_A)
        if j + 1 < n:
          plsc.subcore_barrier()

  return agg(*tables, src16, dst16)


def _sc_agg_edges(table, zeros128, src32, dst32):
  @functools.partial(
      pl.kernel,
      out_type=[jax.ShapeDtypeStruct((NP, 128), f32)] * 2,
      mesh=_mesh,
      scratch_types=[
          pltpu.VMEM((HB, K), jnp.int32),
          pltpu.VMEM((HB, K), jnp.int32),
          pltpu.VMEM((K, 128), f32),
          pltpu.VMEM((K, 128), f32),
          pltpu.VMEM_SHARED((NP, 128), f32),
          pltpu.SemaphoreType.DMA,
          pltpu.SemaphoreType.DMA,
          pltpu.SemaphoreType.DMA,
          pltpu.SemaphoreType.DMA,
      ],
  )
  def agg(t_hbm, z_hbm, src_hbm, dst_hbm, out0, out1,
          src_v, dst_v, buf0, buf1, acc, sem0, sem1, ssem0, ssem1):
    c = lax.axis_index("c")
    s = lax.axis_index("s")
    w = s * 2 + c

    @pl.when(c == 0)
    def _():
      _agg_chunk(t_hbm, t_hbm, out0, acc, src_hbm, dst_hbm, w,
                 src_v, dst_v, buf0, buf1, sem0, sem1, ssem0, ssem1, s, NB_E)

    @pl.when(c == 1)
    def _():
      _agg_chunk(t_hbm, z_hbm, out1, acc, src_hbm, dst_hbm, w,
                 src_v, dst_v, buf0, buf1, sem0, sem1, ssem0, ssem1, s, NB_E)

  return agg(table, zeros128, src32, dst32)


def _tc_pre(histA, histB, xpad):
  blk = 1024

  def body(hA, hB, x_ref, dinv_ref, o0, o1):
    m = pl.program_id(0)
    deg = hA[...][:, :16] + hB[...][:, :16] + 1.0
    rows = lax.broadcasted_iota(jnp.int32, (blk, 16), 0) + m * blk
    dinv = jnp.where(rows < N, lax.rsqrt(deg), 0.0)
    dinv_ref[...] = dinv
    d1 = dinv[:, :1]
    xv = x_ref[...]
    o0[...] = xv[:, :128] * d1
    o1[...] = xv[:, 128:] * d1

  return pl.pallas_call(
      body,
      grid=(NP // blk,),
      in_specs=[
          pl.BlockSpec((blk, 128), lambda m: (m, 0)),
          pl.BlockSpec((blk, 128), lambda m: (m, 0)),
          pl.BlockSpec((blk, D_IN), lambda m: (m, 0)),
      ],
      out_specs=[
          pl.BlockSpec((blk, 16), lambda m: (m, 0)),
          pl.BlockSpec((blk, 128), lambda m: (m, 0)),
          pl.BlockSpec((blk, 128), lambda m: (m, 0)),
      ],
      out_shape=[
          jax.ShapeDtypeStruct((NP, 16), f32),
          jax.ShapeDtypeStruct((NP, 128), f32),
          jax.ShapeDtypeStruct((NP, 128), f32),
      ],
  )(histA, histB, xpad)


def _tc_l1(a0, a1, dinv, W1, b1):
  blk = 256

  def body(r0, r1, dinv_ref, w_ref, b_ref, o0, o1, o2, o3):
    d1 = dinv_ref[...][:, :1]
    z0 = r0[...] * d1
    z1 = r1[...] * d1
    h = (jnp.dot(z0, w_ref[0:128, :], preferred_element_type=f32)
         + jnp.dot(z1, w_ref[128:256, :], preferred_element_type=f32)
         + b_ref[...])
    h = jnp.maximum(h, 0.0) * d1
    o0[...] = h[:, 0:128]
    o1[...] = h[:, 128:256]
    o2[...] = h[:, 256:384]
    o3[...] = h[:, 384:512]

  return pl.pallas_call(
      body,
      grid=(NP // blk,),
      in_specs=[
          pl.BlockSpec((blk, 128), lambda m: (m, 0)),
          pl.BlockSpec((blk, 128), lambda m: (m, 0)),
          pl.BlockSpec((blk, 16), lambda m: (m, 0)),
          pl.BlockSpec((D_IN, D_HID), lambda m: (0, 0)),
          pl.BlockSpec((1, D_HID), lambda m: (0, 0)),
      ],
      out_specs=[pl.BlockSpec((blk, 128), lambda m: (m, 0))] * 4,
      out_shape=[jax.ShapeDtypeStruct((NP, 128), f32)] * 4,
  )(a0, a1, dinv, W1, b1)


def _tc_l2(a0, a1, a2, a3, dinv, W2, b2, W3):
  blk = 256

  def body(r0, r1, r2, r3, dinv_ref, w2_ref, b2_ref, w3_ref, o):
    d1 = dinv_ref[...][:, :1]
    h = b2_ref[...] + jnp.zeros((blk, D_HID), f32)
    for j, r in enumerate((r0, r1, r2, r3)):
      h = h + jnp.dot(r[...] * d1, w2_ref[j * 128:(j + 1) * 128, :],
                      preferred_element_type=f32)
    h = jnp.maximum(h, 0.0)
    g = jnp.dot(h, w3_ref[...], preferred_element_type=f32)
    o[:, 0:D_OUT] = g * d1
    o[:, D_OUT:] = jnp.zeros((blk, 128 - D_OUT), f32)

  return pl.pallas_call(
      body,
      grid=(NP // blk,),
      in_specs=[
          pl.BlockSpec((blk, 128), lambda m: (m, 0)),
          pl.BlockSpec((blk, 128), lambda m: (m, 0)),
          pl.BlockSpec((blk, 128), lambda m: (m, 0)),
          pl.BlockSpec((blk, 128), lambda m: (m, 0)),
          pl.BlockSpec((blk, 16), lambda m: (m, 0)),
          pl.BlockSpec((D_HID, D_HID), lambda m: (0, 0)),
          pl.BlockSpec((1, D_HID), lambda m: (0, 0)),
          pl.BlockSpec((D_HID, D_OUT), lambda m: (0, 0)),
      ],
      out_specs=pl.BlockSpec((blk, 128), lambda m: (m, 0)),
      out_shape=jax.ShapeDtypeStruct((NP, 128), f32),
  )(a0, a1, a2, a3, dinv, W2, b2, W3)


def _tc_out(p0, p1, dinv, b3):
  blk = 1024

  def body(r0, r1, dinv_ref, b_ref, o):
    d1 = dinv_ref[...][:, :1]
    o[...] = (r0[...][:, :D_OUT] + r1[...][:, :D_OUT]) * d1 + b_ref[...]

  return pl.pallas_call(
      body,
      grid=(NP // blk,),
      in_specs=[
          pl.BlockSpec((blk, 128), lambda m: (m, 0)),
          pl.BlockSpec((blk, 128), lambda m: (m, 0)),
          pl.BlockSpec((blk, 16), lambda m: (m, 0)),
          pl.BlockSpec((1, D_OUT), lambda m: (0, 0)),
      ],
      out_specs=pl.BlockSpec((blk, D_OUT), lambda m: (m, 0)),
      out_shape=jax.ShapeDtypeStruct((NP, D_OUT), f32),
  )(p0, p1, dinv, b3)


def kernel(x, edge_index, W1, b1, W2, b2, W3, b3):
  x = x.astype(f32)
  src = edge_index[0].astype(jnp.int32)
  dst = edge_index[1].astype(jnp.int32)
  pad = EP - E
  srcp = jnp.concatenate([src, jnp.zeros((pad,), jnp.int32)])
  dstp = jnp.concatenate([dst, jnp.full((pad,), PAD_ROW, jnp.int32)])
  src16 = srcp.reshape(16, NB_A, K)
  dst16 = dstp.reshape(16, NB_A, K)
  src32 = srcp.reshape(32, NB_E, K)
  dst32 = dstp.reshape(32, NB_E, K)
  xpad = jnp.pad(x, ((0, NP - N), (0, 0)))

  zeros128 = jnp.zeros((NP, 128), f32)
  ones128 = jnp.ones((K, 128), f32)

  hA, hB = _sc_hist(dst32, zeros128, ones128)
  dinv, xt0, xt1 = _tc_pre(hA, hB, xpad)
  g0, g1 = _sc_agg_cols([xt0, xt1], src16, dst16)
  h0, h1, h2, h3 = _tc_l1(g0, g1, dinv, W1, b1.reshape(1, -1))
  a0, a1, a2, a3 = _sc_agg_cols([h0, h1, h2, h3], src16, dst16)
  gt = _tc_l2(a0, a1, a2, a3, dinv, W2, b2.reshape(1, -1), W3)
  p0, p1 = _sc_agg_edges(gt, zeros128, src32, dst32)
  out = _tc_out(p0, p1, dinv, b3.reshape(1, -1))
  return out[:N]


# speedup vs baseline: 1.0562x; 1.0562x over previous
import functools

import jax
import jax.numpy as jnp
from jax import lax
from jax.experimental import pallas as pl
from jax.experimental.pallas import tpu as pltpu
from jax.experimental.pallas import tpu_sc as plsc

N = 10000
NP = 10240
E = 160000
EP = 163840
PAD_ROW = 10000
D_IN = 256
D_HID = 512
D_OUT = 32
K = 128
NB_A = EP // 16 // K
NB_E = EP // 32 // K
RPS = NP // 16

_mesh = plsc.VectorSubcoreMesh(core_axis_name="c", subcore_axis_name="s",
                               num_cores=2, num_subcores=16)

f32 = jnp.float32


def _sc_hist(dst32, zeros128, ones128):
  @functools.partial(
      pl.kernel,
      out_type=jax.ShapeDtypeStruct((2, NP, 128), f32),
      mesh=_mesh,
      scratch_types=[
          pltpu.VMEM((NB_E, K), jnp.int32),
          pltpu.VMEM((K, 128), f32),
          pltpu.VMEM_SHARED((NP, 128), f32),
      ],
  )
  def hist(dst_hbm, z_hbm, ones_hbm, out, dst_v, ones_v, acc):
    c = lax.axis_index("c")
    s = lax.axis_index("s")
    w = s * 2 + c
    rows = pl.ds(s * RPS, RPS)
    pltpu.sync_copy(dst_hbm.at[w], dst_v)
    pltpu.sync_copy(ones_hbm, ones_v)
    pltpu.sync_copy(z_hbm.at[rows], acc.at[rows])
    plsc.subcore_barrier()

    def body(b, carry):
      pltpu.sync_copy(ones_v, acc.at[dst_v.at[b]], add=True)
      return carry

    lax.fori_loop(0, NB_E, body, 0)
    plsc.subcore_barrier()
    pltpu.sync_copy(acc.at[rows], out.at[c, rows])

  h = hist(dst32, zeros128, ones128)
  return h[0], h[1]


HB = 40


def _agg_chunk(t_hbm, init_hbm, o_hbm, acc, src_hbm, dst_hbm, row_i,
               src_v, dst_v, buf0, buf1, sem0, sem1, s, nb):
  rows = pl.ds(s * RPS, RPS)
  pltpu.sync_copy(init_hbm.at[rows], acc.at[rows])
  plsc.subcore_barrier()

  for h in range(nb // HB):
    pltpu.sync_copy(src_hbm.at[row_i, pl.ds(h * HB, HB)], src_v)
    pltpu.sync_copy(dst_hbm.at[row_i, pl.ds(h * HB, HB)], dst_v)
    pltpu.async_copy(t_hbm.at[src_v.at[0]], buf0, sem0)

    def body(i, carry):
      b = 2 * i
      pltpu.async_copy(t_hbm.at[src_v.at[b + 1]], buf1, sem1)
      pltpu.make_async_copy(t_hbm.at[src_v.at[b]], buf0, sem0).wait()
      pltpu.sync_copy(buf0, acc.at[dst_v.at[b]], add=True)

      @pl.when(b + 2 < HB)
      def _():
        pltpu.async_copy(t_hbm.at[src_v.at[b + 2]], buf0, sem0)

      pltpu.make_async_copy(t_hbm.at[src_v.at[b + 1]], buf1, sem1).wait()
      pltpu.sync_copy(buf1, acc.at[dst_v.at[b + 1]], add=True)
      return carry

    lax.fori_loop(0, HB // 2, body, 0)

  plsc.subcore_barrier()
  pltpu.sync_copy(acc.at[rows], o_hbm.at[rows])


def _sc_agg_cols(tables, src16, dst16):
  n = len(tables)
  per_core = n // 2

  @functools.partial(
      pl.kernel,
      out_type=[jax.ShapeDtypeStruct((NP, 128), f32)] * n,
      mesh=_mesh,
      scratch_types=[
          pltpu.VMEM((HB, K), jnp.int32),
          pltpu.VMEM((HB, K), jnp.int32),
          pltpu.VMEM((K, 128), f32),
          pltpu.VMEM((K, 128), f32),
          pltpu.VMEM_SHARED((NP, 128), f32),
          pltpu.SemaphoreType.DMA,
          pltpu.SemaphoreType.DMA,
      ],
  )
  def agg(*refs):
    t_refs = refs[:n]
    src_hbm, dst_hbm = refs[n], refs[n + 1]
    o_refs = refs[n + 2:2 * n + 2]
    src_v, dst_v, buf0, buf1, acc, sem0, sem1 = refs[2 * n + 2:]
    c = lax.axis_index("c")
    s = lax.axis_index("s")

    @pl.when(c == 0)
    def _():
      for j in range(per_core):
        _agg_chunk(t_refs[j], t_refs[j], o_refs[j], acc, src_hbm, dst_hbm, s,
                   src_v, dst_v, buf0, buf1, sem0, sem1, s, NB_A)
        if j + 1 < per_core:
          plsc.subcore_barrier()

    @pl.when(c == 1)
    def _():
      for j in range(per_core, n):
        _agg_chunk(t_refs[j], t_refs[j], o_refs[j], acc, src_hbm, dst_hbm, s,
                   src_v, dst_v, buf0, buf1, sem0, sem1, s, NB_A)
        if j + 1 < n:
          plsc.subcore_barrier()

  return agg(*tables, src16, dst16)


def _sc_agg_edges(table, zeros128, src32, dst32):
  @functools.partial(
      pl.kernel,
      out_type=[jax.ShapeDtypeStruct((NP, 128), f32)] * 2,
      mesh=_mesh,
      scratch_types=[
          pltpu.VMEM((HB, K), jnp.int32),
          pltpu.VMEM((HB, K), jnp.int32),
          pltpu.VMEM((K, 128), f32),
          pltpu.VMEM((K, 128), f32),
          pltpu.VMEM_SHARED((NP, 128), f32),
          pltpu.SemaphoreType.DMA,
          pltpu.SemaphoreType.DMA,
      ],
  )
  def agg(t_hbm, z_hbm, src_hbm, dst_hbm, out0, out1,
          src_v, dst_v, buf0, buf1, acc, sem0, sem1):
    c = lax.axis_index("c")
    s = lax.axis_index("s")
    w = s * 2 + c

    @pl.when(c == 0)
    def _():
      _agg_chunk(t_hbm, t_hbm, out0, acc, src_hbm, dst_hbm, w,
                 src_v, dst_v, buf0, buf1, sem0, sem1, s, NB_E)

    @pl.when(c == 1)
    def _():
      _agg_chunk(t_hbm, z_hbm, out1, acc, src_hbm, dst_hbm, w,
                 src_v, dst_v, buf0, buf1, sem0, sem1, s, NB_E)

  return agg(table, zeros128, src32, dst32)


def _tc_pre(histA, histB, xpad):
  blk = 1024

  def body(hA, hB, x_ref, dinv_ref, o0, o1):
    m = pl.program_id(0)
    deg = hA[...][:, :16] + hB[...][:, :16] + 1.0
    rows = lax.broadcasted_iota(jnp.int32, (blk, 16), 0) + m * blk
    dinv = jnp.where(rows < N, lax.rsqrt(deg), 0.0)
    dinv_ref[...] = dinv
    d1 = dinv[:, :1]
    xv = x_ref[...]
    o0[...] = xv[:, :128] * d1
    o1[...] = xv[:, 128:] * d1

  return pl.pallas_call(
      body,
      grid=(NP // blk,),
      in_specs=[
          pl.BlockSpec((blk, 128), lambda m: (m, 0)),
          pl.BlockSpec((blk, 128), lambda m: (m, 0)),
          pl.BlockSpec((blk, D_IN), lambda m: (m, 0)),
      ],
      out_specs=[
          pl.BlockSpec((blk, 16), lambda m: (m, 0)),
          pl.BlockSpec((blk, 128), lambda m: (m, 0)),
          pl.BlockSpec((blk, 128), lambda m: (m, 0)),
      ],
      out_shape=[
          jax.ShapeDtypeStruct((NP, 16), f32),
          jax.ShapeDtypeStruct((NP, 128), f32),
          jax.ShapeDtypeStruct((NP, 128), f32),
      ],
  )(histA, histB, xpad)


def _tc_l1(a0, a1, dinv, W1, b1):
  blk = 256

  def body(r0, r1, dinv_ref, w_ref, b_ref, o0, o1, o2, o3):
    d1 = dinv_ref[...][:, :1]
    bf = jnp.bfloat16
    z0 = (r0[...] * d1).astype(bf)
    z1 = (r1[...] * d1).astype(bf)
    w = w_ref[...].astype(bf)
    h = (jnp.dot(z0, w[0:128, :], preferred_element_type=f32)
         + jnp.dot(z1, w[128:256, :], preferred_element_type=f32)
         + b_ref[...])
    h = jnp.maximum(h, 0.0) * d1
    o0[...] = h[:, 0:128]
    o1[...] = h[:, 128:256]
    o2[...] = h[:, 256:384]
    o3[...] = h[:, 384:512]

  return pl.pallas_call(
      body,
      grid=(NP // blk,),
      in_specs=[
          pl.BlockSpec((blk, 128), lambda m: (m, 0)),
          pl.BlockSpec((blk, 128), lambda m: (m, 0)),
          pl.BlockSpec((blk, 16), lambda m: (m, 0)),
          pl.BlockSpec((D_IN, D_HID), lambda m: (0, 0)),
          pl.BlockSpec((1, D_HID), lambda m: (0, 0)),
      ],
      out_specs=[pl.BlockSpec((blk, 128), lambda m: (m, 0))] * 4,
      out_shape=[jax.ShapeDtypeStruct((NP, 128), f32)] * 4,
  )(a0, a1, dinv, W1, b1)


def _tc_l2(a0, a1, a2, a3, dinv, W2, b2, W3):
  blk = 256

  def body(r0, r1, r2, r3, dinv_ref, w2_ref, b2_ref, w3_ref, o):
    d1 = dinv_ref[...][:, :1]
    bf = jnp.bfloat16
    w2 = w2_ref[...].astype(bf)
    h = b2_ref[...] + jnp.zeros((blk, D_HID), f32)
    for j, r in enumerate((r0, r1, r2, r3)):
      h = h + jnp.dot((r[...] * d1).astype(bf), w2[j * 128:(j + 1) * 128, :],
                      preferred_element_type=f32)
    h = jnp.maximum(h, 0.0)
    g = jnp.dot(h.astype(bf), w3_ref[...].astype(bf),
                preferred_element_type=f32)
    o[:, 0:D_OUT] = g * d1
    o[:, D_OUT:] = jnp.zeros((blk, 128 - D_OUT), f32)

  return pl.pallas_call(
      body,
      grid=(NP // blk,),
      in_specs=[
          pl.BlockSpec((blk, 128), lambda m: (m, 0)),
          pl.BlockSpec((blk, 128), lambda m: (m, 0)),
          pl.BlockSpec((blk, 128), lambda m: (m, 0)),
          pl.BlockSpec((blk, 128), lambda m: (m, 0)),
          pl.BlockSpec((blk, 16), lambda m: (m, 0)),
          pl.BlockSpec((D_HID, D_HID), lambda m: (0, 0)),
          pl.BlockSpec((1, D_HID), lambda m: (0, 0)),
          pl.BlockSpec((D_HID, D_OUT), lambda m: (0, 0)),
      ],
      out_specs=pl.BlockSpec((blk, 128), lambda m: (m, 0)),
      out_shape=jax.ShapeDtypeStruct((NP, 128), f32),
  )(a0, a1, a2, a3, dinv, W2, b2, W3)


def _tc_out(p0, p1, dinv, b3):
  blk = 1024

  def body(r0, r1, dinv_ref, b_ref, o):
    d1 = dinv_ref[...][:, :1]
    o[...] = (r0[...][:, :D_OUT] + r1[...][:, :D_OUT]) * d1 + b_ref[...]

  return pl.pallas_call(
      body,
      grid=(NP // blk,),
      in_specs=[
          pl.BlockSpec((blk, 128), lambda m: (m, 0)),
          pl.BlockSpec((blk, 128), lambda m: (m, 0)),
          pl.BlockSpec((blk, 16), lambda m: (m, 0)),
          pl.BlockSpec((1, D_OUT), lambda m: (0, 0)),
      ],
      out_specs=pl.BlockSpec((blk, D_OUT), lambda m: (m, 0)),
      out_shape=jax.ShapeDtypeStruct((NP, D_OUT), f32),
  )(p0, p1, dinv, b3)


def kernel(x, edge_index, W1, b1, W2, b2, W3, b3):
  x = x.astype(f32)
  src = edge_index[0].astype(jnp.int32)
  dst = edge_index[1].astype(jnp.int32)
  pad = EP - E
  srcp = jnp.concatenate([src, jnp.zeros((pad,), jnp.int32)])
  dstp = jnp.concatenate([dst, jnp.full((pad,), PAD_ROW, jnp.int32)])
  src16 = srcp.reshape(16, NB_A, K)
  dst16 = dstp.reshape(16, NB_A, K)
  src32 = srcp.reshape(32, NB_E, K)
  dst32 = dstp.reshape(32, NB_E, K)
  xpad = jnp.pad(x, ((0, NP - N), (0, 0)))

  zeros128 = jnp.zeros((NP, 128), f32)
  ones128 = jnp.ones((K, 128), f32)

  hA, hB = _sc_hist(dst32, zeros128, ones128)
  dinv, xt0, xt1 = _tc_pre(hA, hB, xpad)
  g0, g1 = _sc_agg_cols([xt0, xt1], src16, dst16)
  h0, h1, h2, h3 = _tc_l1(g0, g1, dinv, W1, b1.reshape(1, -1))
  a0, a1, a2, a3 = _sc_agg_cols([h0, h1, h2, h3], src16, dst16)
  gt = _tc_l2(a0, a1, a2, a3, dinv, W2, b2.reshape(1, -1), W3)
  p0, p1 = _sc_agg_edges(gt, zeros128, src32, dst32)
  out = _tc_out(p0, p1, dinv, b3.reshape(1, -1))
  return out[:N]
